# baseline (speedup 1.0000x reference)
import jax
import jax.numpy as jnp
from jax.experimental import pallas as pl
from jax.experimental.pallas import tpu as pltpu

_GROUPS = 4


def _fused_kernel(lbl_ref, x_ref, w1_ref, w2_ref, feat_ref, lab_ref):
    xb = x_ref[0].astype(jnp.bfloat16)
    hid = jnp.dot(w1_ref[...], xb, preferred_element_type=jnp.float32)
    hb = jnp.maximum(hid, 0.0).astype(jnp.bfloat16)
    feat_ref[0] = jnp.dot(w2_ref[...], hb, preferred_element_type=jnp.float32)
    b = pl.program_id(0)
    lab_ref[...] = jnp.full(lab_ref.shape, lbl_ref[b], dtype=jnp.int32)


def _conv_group(x_g, w1b, w2b, lbl_g):
    gb, C, HW = x_g.shape
    hidden = w1b.shape[0]
    out_c = w2b.shape[0]
    return pl.pallas_call(
        _fused_kernel,
        out_shape=(
            jax.ShapeDtypeStruct((gb, out_c, HW), jnp.float32),
            jax.ShapeDtypeStruct((gb, 1, HW), jnp.int32),
        ),
        grid_spec=pltpu.PrefetchScalarGridSpec(
            num_scalar_prefetch=1,
            grid=(gb,),
            in_specs=[
                pl.BlockSpec((1, C, HW), lambda b, lbl: (b, 0, 0)),
                pl.BlockSpec((hidden, C), lambda b, lbl: (0, 0)),
                pl.BlockSpec((out_c, hidden), lambda b, lbl: (0, 0)),
            ],
            out_specs=(
                pl.BlockSpec((1, out_c, HW), lambda b, lbl: (b, 0, 0)),
                pl.BlockSpec((1, 1, HW), lambda b, lbl: (b, 0, 0)),
            ),
        ),
        compiler_params=pltpu.CompilerParams(
            dimension_semantics=("parallel",)),
    )(lbl_g, x_g, w1b, w2b)


def kernel(x, w1, w2, need_backprop):
    B, C, H, W = x.shape
    out_c = w2.shape[0]
    HW = H * W

    lbl = need_backprop.astype(jnp.float32).astype(jnp.int32)
    w1b = w1.astype(jnp.bfloat16)
    w2b = w2.astype(jnp.bfloat16)

    groups = _GROUPS if B % _GROUPS == 0 else 1
    gb = B // groups
    feats, labs = [], []
    for g in range(groups):
        x_g = jax.lax.slice_in_dim(x, g * gb, (g + 1) * gb, axis=0)
        x_g = x_g.reshape(gb, C, HW)
        lbl_g = jax.lax.slice_in_dim(lbl, g * gb, (g + 1) * gb, axis=0)
        feat_g, lab_g = _conv_group(x_g, w1b, w2b, lbl_g)
        feats.append(feat_g)
        labs.append(lab_g)

    feat = jnp.concatenate(feats, axis=0) if groups > 1 else feats[0]
    lab = jnp.concatenate(labs, axis=0) if groups > 1 else labs[0]
    return feat.reshape(B, out_c, H, W), lab.reshape(B, H, W)
